# baseline (speedup 1.0000x reference)
import functools
import jax
import jax.numpy as jnp
from jax import lax
from jax.experimental import pallas as pl
from jax.experimental.pallas import tpu as pltpu
from jax.experimental.pallas import tpu_sc as plsc

_S, _B, _D = 8192, 4, 1024
_NC, _NS = 2, 16
_NW = _NC * _NS
_ROWS = _S // _NW
_R = 4
_NCHUNK = _ROWS // _R
_SLOTS = 4
_LANES = 16


def _build():
    mesh = plsc.VectorSubcoreMesh(
        core_axis_name="c", subcore_axis_name="s",
        num_cores=_NC, num_subcores=_NS)

    @functools.partial(
        pl.kernel,
        out_type=jax.ShapeDtypeStruct((_S, _B, _D), jnp.float32),
        mesh=mesh,
        scratch_types=[
            pltpu.VMEM((_SLOTS, _R, _B, _D), jnp.float32),
            pltpu.VMEM((_SLOTS, _R, _D), jnp.float32),
            pltpu.SemaphoreType.DMA((_SLOTS,)),
            pltpu.SemaphoreType.DMA((_SLOTS,)),
        ],
        compiler_params=pltpu.CompilerParams(use_tc_tiling_on_sc=True),
    )
    def sc_add(x_hbm, pe_hbm, out_hbm, xv, pev, insem, outsem):
        wid = lax.axis_index("s") * _NC + lax.axis_index("c")
        base = wid * _ROWS

        def in_copies(i, slot):
            row = base + i * _R
            return (
                pltpu.make_async_copy(
                    x_hbm.at[pl.ds(row, _R)], xv.at[slot], insem.at[slot]),
                pltpu.make_async_copy(
                    pe_hbm.at[pl.ds(row, _R)], pev.at[slot], insem.at[slot]),
            )

        def out_copy(i, slot):
            row = base + i * _R
            return pltpu.make_async_copy(
                xv.at[slot], out_hbm.at[pl.ds(row, _R)], outsem.at[slot])

        def start_in(i, slot):
            a, b = in_copies(i, slot)
            a.start()
            b.start()

        def wait_in(i, slot):
            a, b = in_copies(i, slot)
            a.wait()
            b.wait()

        _LEAD = _SLOTS - 1
        for s in range(_LEAD):
            start_in(s, s)

        def compute(slot):
            @pl.loop(0, _D // _LANES, unroll=2)
            def _(c):
                off = c * _LANES
                for r in range(_R):
                    p = pev[slot, r, pl.ds(off, _LANES)]
                    for q in range(_B):
                        xv[slot, r, q, pl.ds(off, _LANES)] = (
                            xv[slot, r, q, pl.ds(off, _LANES)] + p)

        @pl.loop(0, _NCHUNK, step=_SLOTS)
        def _(g):
            for b in range(_SLOTS):
                i = g + b
                wait_in(i, b)
                compute(b)
                out_copy(i, b).start()
                nslot = (b + _LEAD) % _SLOTS
                nxt = i + _LEAD

                @pl.when(nxt < _NCHUNK)
                def _():
                    @pl.when(i >= 1)
                    def _():
                        out_copy(i - 1, nslot).wait()

                    start_in(nxt, nslot)

        for s in range(_SLOTS):
            out_copy(_NCHUNK - _SLOTS + s, s).wait()

    return sc_add


_sc_impl = _build()


def kernel(x, position_embeddings):
    S = x.shape[0]
    return _sc_impl(x, position_embeddings[:S])


# speedup vs baseline: 2.6695x; 2.6695x over previous
import functools
import jax
import jax.numpy as jnp
from jax import lax
from jax.experimental import pallas as pl
from jax.experimental.pallas import tpu as pltpu
from jax.experimental.pallas import tpu_sc as plsc

_S, _B, _D = 8192, 4, 1024
_NC, _NS = 2, 16
_NW = _NC * _NS
_ROWS = _S // _NW
_R = 4
_NCHUNK = _ROWS // _R
_SLOTS = 4
_LANES = 16


def _build():
    mesh = plsc.VectorSubcoreMesh(
        core_axis_name="c", subcore_axis_name="s",
        num_cores=_NC, num_subcores=_NS)

    @functools.partial(
        pl.kernel,
        out_type=jax.ShapeDtypeStruct((_S, _B, _D), jnp.float32),
        mesh=mesh,
        scratch_types=[
            pltpu.VMEM((_SLOTS, _R, _B, _D), jnp.float32),
            pltpu.VMEM((_SLOTS, _R, _D), jnp.float32),
            pltpu.SemaphoreType.DMA((_SLOTS,)),
            pltpu.SemaphoreType.DMA((_SLOTS,)),
        ],
        compiler_params=pltpu.CompilerParams(use_tc_tiling_on_sc=True),
    )
    def sc_add(x_hbm, pe_hbm, out_hbm, xv, pev, insem, outsem):
        wid = lax.axis_index("s") * _NC + lax.axis_index("c")
        base = wid * _ROWS

        def in_copies(i, slot):
            row = base + i * _R
            return (
                pltpu.make_async_copy(
                    x_hbm.at[pl.ds(row, _R)], xv.at[slot], insem.at[slot]),
                pltpu.make_async_copy(
                    pe_hbm.at[pl.ds(row, _R)], pev.at[slot], insem.at[slot]),
            )

        def out_copy(i, slot):
            row = base + i * _R
            return pltpu.make_async_copy(
                xv.at[slot], out_hbm.at[pl.ds(row, _R)], outsem.at[slot])

        def start_in(i, slot):
            a, b = in_copies(i, slot)
            a.start()
            b.start()

        def wait_in(i, slot):
            a, b = in_copies(i, slot)
            a.wait()
            b.wait()

        _LEAD = _SLOTS - 1
        for s in range(_LEAD):
            start_in(s, s)

        def compute(slot):
            @plsc.parallel_loop(0, _D // _LANES)
            def _(c):
                off = c * _LANES
                for r in range(_R):
                    p = pev[slot, r, pl.ds(off, _LANES)]
                    for q in range(_B):
                        xv[slot, r, q, pl.ds(off, _LANES)] = (
                            xv[slot, r, q, pl.ds(off, _LANES)] + p)

        @pl.loop(0, _NCHUNK, step=_SLOTS)
        def _(g):
            for b in range(_SLOTS):
                i = g + b
                wait_in(i, b)
                compute(b)
                out_copy(i, b).start()
                nslot = (b + _LEAD) % _SLOTS
                nxt = i + _LEAD

                @pl.when(nxt < _NCHUNK)
                def _():
                    @pl.when(i >= 1)
                    def _():
                        out_copy(i - 1, nslot).wait()

                    start_in(nxt, nslot)

        for s in range(_SLOTS):
            out_copy(_NCHUNK - _SLOTS + s, s).wait()

    return sc_add


_sc_impl = _build()


def kernel(x, position_embeddings):
    S = x.shape[0]
    return _sc_impl(x, position_embeddings[:S])
